# baseline (speedup 1.0000x reference)
import functools

import jax
import jax.numpy as jnp
from jax.experimental import pallas as pl
from jax.experimental.pallas import tpu as pltpu

_EPS = 1e-6
_NEG_BIG = float(jnp.finfo(jnp.float32).min)


def _pooler_body(nc, scale,
                 q_ref, tok_ref, valid_ref, lat_ref, wq_ref, bq_ref,
                 wk_ref, bk_ref, wv_ref, bv_ref, nw_ref,
                 out_ref, mask_ref,
                 lq_ref, acc_ref, m_ref, l_ref, hv_ref):
    j = pl.program_id(1)

    @pl.when(j == 0)
    def _init():
        lq = lat_ref[...] + (
            jnp.dot(q_ref[0], wq_ref[...], preferred_element_type=jnp.float32)
            + bq_ref[...])
        lq_ref[...] = lq
        acc_ref[...] = jnp.zeros_like(acc_ref)
        m_ref[...] = jnp.full_like(m_ref, _NEG_BIG)
        l_ref[...] = jnp.zeros_like(l_ref)
        hv_ref[...] = jnp.zeros_like(hv_ref)

    t = tok_ref[0]
    vrow = valid_ref[0, 0]
    hv_ref[...] = jnp.maximum(hv_ref[...], vrow)

    k = jnp.dot(t, wk_ref[...], preferred_element_type=jnp.float32) + bk_ref[...]
    v = jnp.dot(t, wv_ref[...], preferred_element_type=jnp.float32) + bv_ref[...]

    s = jax.lax.dot_general(lq_ref[...], k, (((1,), (1,)), ((), ())),
                            preferred_element_type=jnp.float32) * scale
    s = jnp.where(vrow > 0.0, s, _NEG_BIG)

    m_prev = m_ref[...]
    m_new = jnp.maximum(m_prev, jnp.max(s, axis=1, keepdims=True))
    p = jnp.exp(s - m_new)
    alpha = jnp.exp(m_prev - m_new)
    l_ref[...] = l_ref[...] * alpha + jnp.sum(p, axis=1, keepdims=True)
    acc_ref[...] = acc_ref[...] * alpha + jnp.dot(
        p, v, preferred_element_type=jnp.float32)
    m_ref[...] = m_new

    @pl.when(j == nc - 1)
    def _finalize():
        o = acc_ref[...] / l_ref[...]
        var = jnp.mean(o * o, axis=1, keepdims=True)
        o = o * jax.lax.rsqrt(var + _EPS) * nw_ref[...]
        anyv = jnp.max(hv_ref[...], axis=1, keepdims=True)
        o = o * jnp.where(anyv > 0.0, 1.0, 0.0)
        out_ref[...] = o[None]
        mask_ref[0] = jnp.broadcast_to(
            jnp.where(anyv > 0.0, 0.0, 1.0), mask_ref.shape[1:])


def kernel(query, tokens, padding_mask, latents, Wq, bq, Wk, bk, Wv, bv, norm_w):
    B, N, D = tokens.shape
    L = latents.shape[0]
    BN = 512
    NC = N // BN
    scale = float(D) ** -0.5

    valid = jnp.logical_not(padding_mask).astype(jnp.float32)
    valid = valid.reshape(B, NC, 1, BN)
    query3 = query.reshape(B, 1, D)
    bq2 = bq.reshape(1, D)
    bk2 = bk.reshape(1, D)
    bv2 = bv.reshape(1, D)
    nw2 = norm_w.reshape(1, D)

    out, mask_f = pl.pallas_call(
        functools.partial(_pooler_body, NC, scale),
        grid=(B, NC),
        in_specs=[
            pl.BlockSpec((1, 1, D), lambda i, j: (i, 0, 0)),
            pl.BlockSpec((1, BN, D), lambda i, j: (i, j, 0)),
            pl.BlockSpec((1, 1, 1, BN), lambda i, j: (i, j, 0, 0)),
            pl.BlockSpec((L, D), lambda i, j: (0, 0)),
            pl.BlockSpec((D, D), lambda i, j: (0, 0)),
            pl.BlockSpec((1, D), lambda i, j: (0, 0)),
            pl.BlockSpec((D, D), lambda i, j: (0, 0)),
            pl.BlockSpec((1, D), lambda i, j: (0, 0)),
            pl.BlockSpec((D, D), lambda i, j: (0, 0)),
            pl.BlockSpec((1, D), lambda i, j: (0, 0)),
            pl.BlockSpec((1, D), lambda i, j: (0, 0)),
        ],
        out_shape=[
            jax.ShapeDtypeStruct((B, L, D), jnp.float32),
            jax.ShapeDtypeStruct((B, 1, L), jnp.float32),
        ],
        out_specs=[
            pl.BlockSpec((1, L, D), lambda i, j: (i, 0, 0)),
            pl.BlockSpec((1, 1, L), lambda i, j: (i, 0, 0)),
        ],
        scratch_shapes=[
            pltpu.VMEM((L, D), jnp.float32),
            pltpu.VMEM((L, D), jnp.float32),
            pltpu.VMEM((L, 1), jnp.float32),
            pltpu.VMEM((L, 1), jnp.float32),
            pltpu.VMEM((1, BN), jnp.float32),
        ],
        compiler_params=pltpu.CompilerParams(
            dimension_semantics=("parallel", "arbitrary"),
        ),
        name="latent_pooler",
    )(query3, tokens, valid, latents, Wq, bq2, Wk, bk2, Wv, bv2, nw2)

    return out, mask_f.reshape(B, L).astype(jnp.bool_)
